# Initial kernel scaffold; baseline (speedup 1.0000x reference)
import jax
import jax.numpy as jnp
from jax.experimental import pallas as pl


def kernel(attention_scores, relative_position_bias_table):
    raise NotImplementedError("write your pallas kernel here")


# speedup vs baseline: 2.0516x; 2.0516x over previous
import functools

import numpy as np
import jax
import jax.numpy as jnp
from jax import lax
from jax.experimental import pallas as pl
from jax.experimental.pallas import tpu as pltpu
from jax.experimental.pallas import tpu_sc as plsc


def _rel_pos_index(height, width):
    cls_len = 3
    num_rel = (2 * height - 1) * (2 * width - 1) + cls_len
    xx, yy = np.meshgrid(np.arange(height), np.arange(width))
    coords = np.stack([yy, xx], axis=-1).reshape(-1, 2)
    rel = coords[:, None, :] - coords[None, :, :]
    rx = (rel[:, :, 0] + width - 1) * (2 * height - 1)
    ry = rel[:, :, 1] + height - 1
    idx = (rx + ry).astype(np.int64)
    top = np.full((1, idx.shape[1]), num_rel - 3, dtype=idx.dtype)
    left = np.full((idx.shape[0], 1), num_rel - 2, dtype=idx.dtype)
    corner = np.full((1, 1), num_rel - 1, dtype=idx.dtype)
    idx = np.concatenate([top, idx], axis=0)
    left_corner = np.concatenate([corner, left], axis=0)
    idx = np.concatenate([left_corner, idx], axis=1)
    return idx.astype(np.int32), num_rel


@functools.partial(jax.jit, static_argnums=(2, 3, 4, 5, 6, 7))
def _sc_gather(table, idx_flat, num_heads, num_rel, f_pad, per_w, sub_e, n_sub):
    info = plsc.get_sparse_core_info()
    num_cores = info.num_cores
    sub_v = sub_e // 16
    mesh = plsc.VectorSubcoreMesh(core_axis_name="c", subcore_axis_name="s")

    @functools.partial(
        pl.kernel,
        mesh=mesh,
        out_type=jax.ShapeDtypeStruct((num_heads * f_pad,), jnp.float32),
        compiler_params=pltpu.CompilerParams(needs_layout_passes=False),
        scratch_types=[
            pltpu.VMEM((num_heads * num_rel,), jnp.float32),
            pltpu.VMEM((per_w,), jnp.int32),
            pltpu.VMEM((num_heads * sub_e,), jnp.float32),
        ],
    )
    def gather_k(table_hbm, idx_hbm, out_hbm, table_v, idx_v, buf_v):
        wid = lax.axis_index("s") * num_cores + lax.axis_index("c")
        base = wid * per_w
        pltpu.sync_copy(table_hbm, table_v)
        pltpu.sync_copy(idx_hbm.at[pl.ds(base, per_w)], idx_v)
        for sub in range(n_sub):
            def body(i, carry):
                iv = idx_v[pl.ds(sub * sub_e + i * 16, 16)]
                for h in range(num_heads):
                    buf_v[pl.ds(h * sub_e + i * 16, 16)] = plsc.load_gather(
                        table_v, [iv + (h * num_rel)])
                return carry
            lax.fori_loop(0, sub_v, body, 0)
            for h in range(num_heads):
                pltpu.sync_copy(
                    buf_v.at[pl.ds(h * sub_e, sub_e)],
                    out_hbm.at[pl.ds(h * f_pad + base + sub * sub_e, sub_e)])

    return gather_k(table.reshape(-1), idx_flat)


def _add_kernel(pos_ref, attn_ref, out_ref):
    out_ref[0, 0] = attn_ref[0, 0] + pos_ref[0]


def kernel(attention_scores, relative_position_bias_table):
    B, H, S, _ = attention_scores.shape
    num_heads, num_rel = relative_position_bias_table.shape
    height = width = int(np.sqrt(S - 1))
    idx_np, _ = _rel_pos_index(height, width)
    f = S * S

    info = plsc.get_sparse_core_info()
    nw = info.num_cores * info.num_subcores
    n_sub = 4
    align = nw * 16 * n_sub
    f_pad = ((f + align - 1) // align) * align
    per_w = f_pad // nw
    sub_e = per_w // n_sub

    idx_flat = np.zeros((f_pad,), dtype=np.int32)
    idx_flat[:f] = idx_np.reshape(-1)
    idx_flat = jnp.asarray(idx_flat)

    pos_flat = _sc_gather(relative_position_bias_table, idx_flat,
                          num_heads, num_rel, f_pad, per_w, sub_e, n_sub)
    pos = pos_flat.reshape(num_heads, f_pad)[:, :f].reshape(num_heads, S, S)

    out = pl.pallas_call(
        _add_kernel,
        grid=(H, B),
        in_specs=[
            pl.BlockSpec((1, S, S), lambda h, b: (h, 0, 0)),
            pl.BlockSpec((1, 1, S, S), lambda h, b: (b, h, 0, 0)),
        ],
        out_specs=pl.BlockSpec((1, 1, S, S), lambda h, b: (b, h, 0, 0)),
        out_shape=jax.ShapeDtypeStruct((B, H, S, S), jnp.float32),
        compiler_params=pltpu.CompilerParams(
            dimension_semantics=("arbitrary", "arbitrary")),
    )(pos, attention_scores)
    return out


# speedup vs baseline: 2.1784x; 1.0618x over previous
import functools

import numpy as np
import jax
import jax.numpy as jnp
from jax import lax
from jax.experimental import pallas as pl
from jax.experimental.pallas import tpu as pltpu
from jax.experimental.pallas import tpu_sc as plsc


def _rel_pos_index(height, width):
    cls_len = 3
    num_rel = (2 * height - 1) * (2 * width - 1) + cls_len
    xx, yy = np.meshgrid(np.arange(height), np.arange(width))
    coords = np.stack([yy, xx], axis=-1).reshape(-1, 2)
    rel = coords[:, None, :] - coords[None, :, :]
    rx = (rel[:, :, 0] + width - 1) * (2 * height - 1)
    ry = rel[:, :, 1] + height - 1
    idx = (rx + ry).astype(np.int64)
    top = np.full((1, idx.shape[1]), num_rel - 3, dtype=idx.dtype)
    left = np.full((idx.shape[0], 1), num_rel - 2, dtype=idx.dtype)
    corner = np.full((1, 1), num_rel - 1, dtype=idx.dtype)
    idx = np.concatenate([top, idx], axis=0)
    left_corner = np.concatenate([corner, left], axis=0)
    idx = np.concatenate([left_corner, idx], axis=1)
    return idx.astype(np.int32), num_rel


@functools.partial(jax.jit, static_argnums=(2, 3, 4, 5, 6, 7))
def _sc_gather(table, idx_flat, num_heads, num_rel, f_pad, per_w, sub_e, n_sub):
    info = plsc.get_sparse_core_info()
    num_cores = info.num_cores
    sub_v = sub_e // 16
    mesh = plsc.VectorSubcoreMesh(core_axis_name="c", subcore_axis_name="s")

    @functools.partial(
        pl.kernel,
        mesh=mesh,
        out_type=jax.ShapeDtypeStruct((num_heads * f_pad,), jnp.float32),
        compiler_params=pltpu.CompilerParams(needs_layout_passes=False),
        scratch_types=[
            pltpu.VMEM((num_heads * num_rel,), jnp.float32),
            pltpu.VMEM((per_w,), jnp.int32),
            pltpu.VMEM((num_heads * sub_e,), jnp.float32),
        ],
    )
    def gather_k(table_hbm, idx_hbm, out_hbm, table_v, idx_v, buf_v):
        wid = lax.axis_index("s") * num_cores + lax.axis_index("c")
        base = wid * per_w
        pltpu.sync_copy(table_hbm, table_v)
        pltpu.sync_copy(idx_hbm.at[pl.ds(base, per_w)], idx_v)
        for sub in range(n_sub):
            def body(i, carry):
                iv = idx_v[pl.ds(sub * sub_e + i * 16, 16)]
                for h in range(num_heads):
                    buf_v[pl.ds(h * sub_e + i * 16, 16)] = plsc.load_gather(
                        table_v, [iv + (h * num_rel)])
                return carry
            lax.fori_loop(0, sub_v, body, 0)
            for h in range(num_heads):
                pltpu.sync_copy(
                    buf_v.at[pl.ds(h * sub_e, sub_e)],
                    out_hbm.at[pl.ds(h * f_pad + base + sub * sub_e, sub_e)])

    return gather_k(table.reshape(-1), idx_flat)


def _add_kernel(pos_ref, attn_ref, out_ref):
    out_ref[:, 0] = attn_ref[:, 0] + pos_ref[:]


def kernel(attention_scores, relative_position_bias_table):
    B, H, S, _ = attention_scores.shape
    num_heads, num_rel = relative_position_bias_table.shape
    height = width = int(np.sqrt(S - 1))
    idx_np, _ = _rel_pos_index(height, width)
    f = S * S

    info = plsc.get_sparse_core_info()
    nw = info.num_cores * info.num_subcores
    n_sub = 4
    align = nw * 16 * n_sub
    f_pad = ((f + align - 1) // align) * align
    per_w = f_pad // nw
    sub_e = per_w // n_sub

    idx_flat = np.zeros((f_pad,), dtype=np.int32)
    idx_flat[:f] = idx_np.reshape(-1)
    idx_flat = jnp.asarray(idx_flat)

    pos_flat = _sc_gather(relative_position_bias_table, idx_flat,
                          num_heads, num_rel, f_pad, per_w, sub_e, n_sub)
    pos = pos_flat.reshape(num_heads, f_pad)[:, :f].reshape(num_heads, S, S)

    bb = 4
    out = pl.pallas_call(
        _add_kernel,
        grid=(H, B // bb),
        in_specs=[
            pl.BlockSpec((1, S, S), lambda h, b: (h, 0, 0)),
            pl.BlockSpec((bb, 1, S, S), lambda h, b: (b, h, 0, 0)),
        ],
        out_specs=pl.BlockSpec((bb, 1, S, S), lambda h, b: (b, h, 0, 0)),
        out_shape=jax.ShapeDtypeStruct((B, H, S, S), jnp.float32),
        compiler_params=pltpu.CompilerParams(
            dimension_semantics=("arbitrary", "arbitrary"),
            vmem_limit_bytes=100 * 1024 * 1024),
    )(pos, attention_scores)
    return out
